# Initial kernel scaffold; baseline (speedup 1.0000x reference)
import jax
import jax.numpy as jnp
from jax.experimental import pallas as pl


def kernel(pred_logits, gt_labels):
    raise NotImplementedError("write your pallas kernel here")


# speedup vs baseline: 1.2464x; 1.2464x over previous
import functools

import jax
import jax.numpy as jnp
from jax import lax
from jax.experimental import pallas as pl
from jax.experimental.pallas import tpu as pltpu
from jax.experimental.pallas import tpu_sc as plsc

_C = 21
_NBINS = 448
_ROWS_BLK = 8192
_NW = 32
_CHUNK = 16384
_LANES = 16


def _bins_body(x_ref, gt_ref, out_ref):
    x = x_ref[...]
    col = lax.broadcasted_iota(jnp.int32, x.shape, 1)
    xm = jnp.where(col == 0, -jnp.inf, x)
    m = jnp.max(xm, axis=1, keepdims=True)
    pred = jnp.min(jnp.where(xm == m, col, _C), axis=1)
    out_ref[...] = pred * _C + gt_ref[...]


def _compute_bins(logits, gt):
    n = logits.shape[0]
    nblk = n // _ROWS_BLK
    return pl.pallas_call(
        _bins_body,
        grid=(nblk,),
        in_specs=[
            pl.BlockSpec((_ROWS_BLK, _C), lambda i: (i, 0)),
            pl.BlockSpec((_ROWS_BLK,), lambda i: (i,)),
        ],
        out_specs=pl.BlockSpec((_ROWS_BLK,), lambda i: (i,)),
        out_shape=jax.ShapeDtypeStruct((n, ), jnp.int32),
    )(logits, gt)


@functools.cache
def _make_hist(n):
    per_w = n // _NW
    n_chunks = per_w // _CHUNK
    mesh = plsc.VectorSubcoreMesh(core_axis_name="c", subcore_axis_name="s")

    @functools.partial(
        pl.kernel,
        out_type=jax.ShapeDtypeStruct((_NW, _NBINS), jnp.float32),
        mesh=mesh,
        compiler_params=pltpu.CompilerParams(needs_layout_passes=False),
        scratch_types=[
            pltpu.VMEM((_CHUNK,), jnp.int32),
            pltpu.VMEM((_LANES * _NBINS,), jnp.float32),
            pltpu.VMEM((_NBINS,), jnp.float32),
        ],
    )
    def hist(bins_hbm, out_hbm, chunk_v, hist_v, part_v):
        cid = lax.axis_index("c")
        sid = lax.axis_index("s")
        wid = sid * 2 + cid
        base = wid * per_w
        lane = lax.iota(jnp.int32, _LANES)
        zeros16 = jnp.zeros((_LANES,), jnp.float32)
        ones16 = jnp.ones((_LANES,), jnp.float32)

        def zinit(i, carry):
            hist_v[pl.ds(i * _LANES, _LANES)] = zeros16
            return carry

        lax.fori_loop(0, (_LANES * _NBINS) // _LANES, zinit, 0)

        def chunk_body(t, carry):
            pltpu.sync_copy(bins_hbm.at[pl.ds(base + t * _CHUNK, _CHUNK)],
                            chunk_v)

            def inner(i, c):
                b = chunk_v[pl.ds(i * _LANES, _LANES)]
                idx = lane * _NBINS + b
                plsc.addupdate_scatter(hist_v, [idx], ones16)
                return c

            lax.fori_loop(0, _CHUNK // _LANES, inner, 0)
            return carry

        lax.fori_loop(0, n_chunks, chunk_body, 0)

        for jb in range(_NBINS // _LANES):
            acc = zeros16
            for l in range(_LANES):
                acc = acc + hist_v[pl.ds(l * _NBINS + jb * _LANES, _LANES)]
            part_v[pl.ds(jb * _LANES, _LANES)] = acc
        pltpu.sync_copy(part_v, out_hbm.at[wid])

    return hist


def kernel(pred_logits, gt_labels):
    bins = _compute_bins(pred_logits, gt_labels.astype(jnp.int32))
    parts = _make_hist(pred_logits.shape[0])(bins)
    return parts.sum(axis=0)[: _C * _C].reshape(_C, _C)


# speedup vs baseline: 7.9816x; 6.4037x over previous
import functools

import jax
import jax.numpy as jnp
from jax import lax
from jax.experimental import pallas as pl
from jax.experimental.pallas import tpu as pltpu
from jax.experimental.pallas import tpu_sc as plsc

_C = 21
_NBINS = 448
_BM = 256
_NW = 32
_CROWS = 128
_LANES = 16


def _bins_body(xt_ref, gt_ref, out_ref):
    m = xt_ref[1]
    for c in range(2, _C):
        m = jnp.maximum(m, xt_ref[c])
    penc = jnp.where(xt_ref[_C - 1] == m, (_C - 1) * _C, 0)
    for c in range(_C - 2, 0, -1):
        penc = jnp.where(xt_ref[c] == m, c * _C, penc)
    out_ref[...] = penc + gt_ref[...]


def _compute_bins(xt, gt2):
    nrows = xt.shape[1]
    nblk = nrows // _BM
    return pl.pallas_call(
        _bins_body,
        grid=(nblk,),
        in_specs=[
            pl.BlockSpec((_C, _BM, 128), lambda i: (0, i, 0)),
            pl.BlockSpec((_BM, 128), lambda i: (i, 0)),
        ],
        out_specs=pl.BlockSpec((_BM, 128), lambda i: (i, 0)),
        out_shape=jax.ShapeDtypeStruct((nrows, 128), jnp.int32),
    )(xt, gt2)


@functools.cache
def _make_hist(nrows):
    rows_per_w = nrows // _NW
    n_chunks = rows_per_w // _CROWS
    mesh = plsc.VectorSubcoreMesh(core_axis_name="c", subcore_axis_name="s")

    @functools.partial(
        pl.kernel,
        out_type=jax.ShapeDtypeStruct((_NW, _NBINS), jnp.float32),
        mesh=mesh,
        compiler_params=pltpu.CompilerParams(needs_layout_passes=False),
        scratch_types=[
            pltpu.VMEM((_CROWS, 128), jnp.int32),
            pltpu.VMEM((_LANES * _NBINS,), jnp.float32),
            pltpu.VMEM((_NBINS,), jnp.float32),
        ],
    )
    def hist(bins_hbm, out_hbm, chunk_v, hist_v, part_v):
        cid = lax.axis_index("c")
        sid = lax.axis_index("s")
        wid = sid * 2 + cid
        row_base = wid * rows_per_w
        lane = lax.iota(jnp.int32, _LANES)
        zeros16 = jnp.zeros((_LANES,), jnp.float32)
        ones16 = jnp.ones((_LANES,), jnp.float32)

        def zinit(i, carry):
            hist_v[pl.ds(i * _LANES, _LANES)] = zeros16
            return carry

        lax.fori_loop(0, (_LANES * _NBINS) // _LANES, zinit, 0)

        def chunk_body(t, carry):
            pltpu.sync_copy(
                bins_hbm.at[pl.ds(row_base + t * _CROWS, _CROWS)], chunk_v)

            def inner(r, c):
                for j in range(128 // _LANES):
                    b = chunk_v[r, pl.ds(j * _LANES, _LANES)]
                    idx = lane * _NBINS + b
                    plsc.addupdate_scatter(hist_v, [idx], ones16)
                return c

            lax.fori_loop(0, _CROWS, inner, 0)
            return carry

        lax.fori_loop(0, n_chunks, chunk_body, 0)

        for jb in range(_NBINS // _LANES):
            acc = zeros16
            for l in range(_LANES):
                acc = acc + hist_v[pl.ds(l * _NBINS + jb * _LANES, _LANES)]
            part_v[pl.ds(jb * _LANES, _LANES)] = acc
        pltpu.sync_copy(part_v, out_hbm.at[wid])

    return hist


def kernel(pred_logits, gt_labels):
    n = pred_logits.shape[0]
    nrows = n // 128
    xt = lax.transpose(pred_logits.reshape(nrows, 128, _C), (2, 0, 1))
    gt2 = gt_labels.astype(jnp.int32).reshape(nrows, 128)
    bins2 = _compute_bins(xt, gt2)
    parts = _make_hist(nrows)(bins2)
    return parts.sum(axis=0)[: _C * _C].reshape(_C, _C)


# speedup vs baseline: 19.0601x; 2.3880x over previous
import functools

import jax
import jax.numpy as jnp
from jax import lax
from jax.experimental import pallas as pl
from jax.experimental.pallas import tpu as pltpu
from jax.experimental.pallas import tpu_sc as plsc

_C = 21
_NBINS = 448
_BL = 32768
_NW = 32
_CHUNK = 16384
_LANES = 16


def _bins_body(xt_ref, gt_ref, out_ref):
    x = xt_ref[...]
    sub = lax.broadcasted_iota(jnp.int32, x.shape, 0)
    xm = jnp.where(sub == 0, -jnp.inf, x)
    m = jnp.max(xm, axis=0)
    pred21 = jnp.min(jnp.where(xm == m[None, :], sub * _C, _C * _C), axis=0)
    out_ref[...] = pred21 + gt_ref[...]


def _compute_bins(xt, gt):
    n = xt.shape[1]
    nblk = n // _BL
    return pl.pallas_call(
        _bins_body,
        grid=(nblk,),
        in_specs=[
            pl.BlockSpec((_C, _BL), lambda i: (0, i)),
            pl.BlockSpec((_BL,), lambda i: (i,)),
        ],
        out_specs=pl.BlockSpec((_BL,), lambda i: (i,)),
        out_shape=jax.ShapeDtypeStruct((n,), jnp.int32),
    )(xt, gt)


@functools.cache
def _make_hist(n):
    per_w = n // _NW
    n_chunks = per_w // _CHUNK
    mesh = plsc.VectorSubcoreMesh(core_axis_name="c", subcore_axis_name="s")

    @functools.partial(
        pl.kernel,
        out_type=jax.ShapeDtypeStruct((_NW, _NBINS), jnp.float32),
        mesh=mesh,
        compiler_params=pltpu.CompilerParams(needs_layout_passes=False),
        scratch_types=[
            pltpu.VMEM((_CHUNK,), jnp.int32),
            pltpu.VMEM((_LANES * _NBINS,), jnp.float32),
            pltpu.VMEM((_NBINS,), jnp.float32),
        ],
    )
    def hist(bins_hbm, out_hbm, chunk_v, hist_v, part_v):
        cid = lax.axis_index("c")
        sid = lax.axis_index("s")
        wid = sid * 2 + cid
        base = wid * per_w
        lane = lax.iota(jnp.int32, _LANES)
        zeros16 = jnp.zeros((_LANES,), jnp.float32)
        ones16 = jnp.ones((_LANES,), jnp.float32)

        def zinit(i, carry):
            hist_v[pl.ds(i * _LANES, _LANES)] = zeros16
            return carry

        lax.fori_loop(0, (_LANES * _NBINS) // _LANES, zinit, 0)

        def chunk_body(t, carry):
            pltpu.sync_copy(bins_hbm.at[pl.ds(base + t * _CHUNK, _CHUNK)],
                            chunk_v)

            def inner(i, c):
                b = chunk_v[pl.ds(i * _LANES, _LANES)]
                idx = lane * _NBINS + b
                plsc.addupdate_scatter(hist_v, [idx], ones16)
                return c

            lax.fori_loop(0, _CHUNK // _LANES, inner, 0)
            return carry

        lax.fori_loop(0, n_chunks, chunk_body, 0)

        for jb in range(_NBINS // _LANES):
            acc = zeros16
            for l in range(_LANES):
                acc = acc + hist_v[pl.ds(l * _NBINS + jb * _LANES, _LANES)]
            part_v[pl.ds(jb * _LANES, _LANES)] = acc
        pltpu.sync_copy(part_v, out_hbm.at[wid])

    return hist


def kernel(pred_logits, gt_labels):
    xt = jnp.transpose(pred_logits)
    bins = _compute_bins(xt, gt_labels.astype(jnp.int32))
    parts = _make_hist(pred_logits.shape[0])(bins)
    return parts.sum(axis=0)[: _C * _C].reshape(_C, _C)


# speedup vs baseline: 21.5917x; 1.1328x over previous
import functools

import jax
import jax.numpy as jnp
from jax import lax
from jax.experimental import pallas as pl
from jax.experimental.pallas import tpu as pltpu
from jax.experimental.pallas import tpu_sc as plsc

_C = 21
_NBINS = 448
_BL = 65536
_NW = 32
_CHUNK = 16384
_LANES = 16


def _bins_body(xt_ref, gt_ref, out_ref):
    x = xt_ref[...]
    sub = lax.broadcasted_iota(jnp.int32, x.shape, 0)
    xm = jnp.where(sub == 0, -jnp.inf, x)
    m = jnp.max(xm, axis=0)
    pred21 = jnp.min(jnp.where(xm == m[None, :], sub * _C, _C * _C), axis=0)
    out_ref[...] = pred21 + gt_ref[...]


def _compute_bins(xt, gt):
    n = xt.shape[1]
    nblk = n // _BL
    return pl.pallas_call(
        _bins_body,
        grid=(nblk,),
        in_specs=[
            pl.BlockSpec((_C, _BL), lambda i: (0, i)),
            pl.BlockSpec((_BL,), lambda i: (i,)),
        ],
        out_specs=pl.BlockSpec((_BL,), lambda i: (i,)),
        out_shape=jax.ShapeDtypeStruct((n,), jnp.int32),
    )(xt, gt)


@functools.cache
def _make_hist(n):
    per_w = n // _NW
    n_chunks = per_w // _CHUNK
    mesh = plsc.VectorSubcoreMesh(core_axis_name="c", subcore_axis_name="s")

    @functools.partial(
        pl.kernel,
        out_type=jax.ShapeDtypeStruct((_NW, _NBINS), jnp.float32),
        mesh=mesh,
        compiler_params=pltpu.CompilerParams(needs_layout_passes=False),
        scratch_types=[
            pltpu.VMEM((2, _CHUNK), jnp.int32),
            pltpu.VMEM((_LANES * _NBINS,), jnp.float32),
            pltpu.VMEM((_NBINS,), jnp.float32),
            pltpu.SemaphoreType.DMA,
            pltpu.SemaphoreType.DMA,
        ],
    )
    def hist(bins_hbm, out_hbm, chunk_v, hist_v, part_v, sem0, sem1):
        cid = lax.axis_index("c")
        sid = lax.axis_index("s")
        wid = sid * 2 + cid
        base = wid * per_w
        lane = lax.iota(jnp.int32, _LANES)
        zeros16 = jnp.zeros((_LANES,), jnp.float32)
        ones16 = jnp.ones((_LANES,), jnp.float32)
        sems = (sem0, sem1)

        def zinit(i, carry):
            hist_v[pl.ds(i * _LANES, _LANES)] = zeros16
            return carry

        lax.fori_loop(0, (_LANES * _NBINS) // _LANES, zinit, 0)

        _U = 8
        handles = [None, None]
        handles[0] = pltpu.async_copy(
            bins_hbm.at[pl.ds(base, _CHUNK)], chunk_v.at[0], sems[0])
        for t in range(n_chunks):
            tb = t % 2
            if t + 1 < n_chunks:
                nb = (t + 1) % 2
                handles[nb] = pltpu.async_copy(
                    bins_hbm.at[pl.ds(base + (t + 1) * _CHUNK, _CHUNK)],
                    chunk_v.at[nb], sems[nb])
            handles[tb].wait()

            def inner(i, c, tb=tb):
                for u in range(_U):
                    b = chunk_v[tb, pl.ds(i * (_U * _LANES) + u * _LANES,
                                          _LANES)]
                    idx = lane * _NBINS + b
                    plsc.addupdate_scatter(hist_v, [idx], ones16)
                return c

            lax.fori_loop(0, _CHUNK // (_U * _LANES), inner, 0)

        for jb in range(_NBINS // _LANES):
            acc = zeros16
            for l in range(_LANES):
                acc = acc + hist_v[pl.ds(l * _NBINS + jb * _LANES, _LANES)]
            part_v[pl.ds(jb * _LANES, _LANES)] = acc
        pltpu.sync_copy(part_v, out_hbm.at[wid])

    return hist


def kernel(pred_logits, gt_labels):
    xt = jnp.transpose(pred_logits)
    bins = _compute_bins(xt, gt_labels.astype(jnp.int32))
    parts = _make_hist(pred_logits.shape[0])(bins)
    return parts.sum(axis=0)[: _C * _C].reshape(_C, _C)


# speedup vs baseline: 21.7090x; 1.0054x over previous
import functools

import jax
import jax.numpy as jnp
from jax import lax
from jax.experimental import pallas as pl
from jax.experimental.pallas import tpu as pltpu
from jax.experimental.pallas import tpu_sc as plsc

_C = 21
_NBINS = 448
_BL = 65536
_NW = 32
_CHUNK = 16384
_LANES = 16


def _bins_body(xt_ref, gt_ref, out_ref):
    x = xt_ref[...]
    sub = lax.broadcasted_iota(jnp.int32, x.shape, 0)
    xm = jnp.where(sub == 0, -jnp.inf, x)
    m = jnp.max(xm, axis=0)
    pred21 = jnp.min(jnp.where(xm == m[None, :], sub * _C, _C * _C), axis=0)
    out_ref[...] = pred21 + gt_ref[...]


def _compute_bins(xt, gt):
    n = xt.shape[1]
    nblk = n // _BL
    return pl.pallas_call(
        _bins_body,
        grid=(nblk,),
        in_specs=[
            pl.BlockSpec((_C, _BL), lambda i: (0, i)),
            pl.BlockSpec((_BL,), lambda i: (i,)),
        ],
        out_specs=pl.BlockSpec((_BL,), lambda i: (i,)),
        out_shape=jax.ShapeDtypeStruct((n,), jnp.int32),
    )(xt, gt)


@functools.cache
def _make_hist(n):
    per_w = n // _NW
    n_chunks = per_w // _CHUNK
    mesh = plsc.VectorSubcoreMesh(core_axis_name="c", subcore_axis_name="s")

    @functools.partial(
        pl.kernel,
        out_type=jax.ShapeDtypeStruct((_NW, _NBINS), jnp.float32),
        mesh=mesh,
        compiler_params=pltpu.CompilerParams(needs_layout_passes=False),
        scratch_types=[
            pltpu.VMEM((2, _CHUNK), jnp.int32),
            pltpu.VMEM((_LANES * _NBINS,), jnp.float32),
            pltpu.VMEM((_NBINS,), jnp.float32),
            pltpu.SemaphoreType.DMA,
            pltpu.SemaphoreType.DMA,
        ],
    )
    def hist(bins_hbm, out_hbm, chunk_v, hist_v, part_v, sem0, sem1):
        cid = lax.axis_index("c")
        sid = lax.axis_index("s")
        wid = sid * 2 + cid
        base = wid * per_w
        lane = lax.iota(jnp.int32, _LANES)
        zeros16 = jnp.zeros((_LANES,), jnp.float32)
        ones16 = jnp.ones((_LANES,), jnp.float32)
        sems = (sem0, sem1)

        def zinit(i, carry):
            hist_v[pl.ds(i * _LANES, _LANES)] = zeros16
            return carry

        lax.fori_loop(0, (_LANES * _NBINS) // _LANES, zinit, 0)

        _U = 8
        handles = [None, None]
        handles[0] = pltpu.async_copy(
            bins_hbm.at[pl.ds(base, _CHUNK)], chunk_v.at[0], sems[0])
        for t in range(n_chunks):
            tb = t % 2
            if t + 1 < n_chunks:
                nb = (t + 1) % 2
                handles[nb] = pltpu.async_copy(
                    bins_hbm.at[pl.ds(base + (t + 1) * _CHUNK, _CHUNK)],
                    chunk_v.at[nb], sems[nb])
            handles[tb].wait()

            def inner(i, c, tb=tb):
                for u in range(_U):
                    b = chunk_v[tb, pl.ds(i * (_U * _LANES) + u * _LANES,
                                          _LANES)]
                    idx = b * _LANES + lane
                    plsc.addupdate_scatter(hist_v, [idx], ones16)
                return c

            lax.fori_loop(0, _CHUNK // (_U * _LANES), inner, 0)

        gidx = lane * _LANES
        for jb in range(_NBINS // _LANES):
            acc = zeros16
            for l in range(_LANES):
                g = plsc.load_gather(hist_v,
                                     [gidx + (jb * _LANES * _LANES + l)])
                acc = acc + g
            part_v[pl.ds(jb * _LANES, _LANES)] = acc
        pltpu.sync_copy(part_v, out_hbm.at[wid])

    return hist


def kernel(pred_logits, gt_labels):
    xt = jnp.transpose(pred_logits)
    bins = _compute_bins(xt, gt_labels.astype(jnp.int32))
    parts = _make_hist(pred_logits.shape[0])(bins)
    return parts.sum(axis=0)[: _C * _C].reshape(_C, _C)


# speedup vs baseline: 26.4998x; 1.2207x over previous
import functools

import jax
import jax.numpy as jnp
from jax import lax
from jax.experimental import pallas as pl
from jax.experimental.pallas import tpu as pltpu
from jax.experimental.pallas import tpu_sc as plsc

_C = 21
_NBINS = 448
_BL = 65536
_NW = 32
_CHUNK = 16384
_LANES = 16


def _bins_body(xt_ref, gt_ref, out_ref):
    x = xt_ref[...]
    sub = lax.broadcasted_iota(jnp.int32, x.shape, 0)
    xm = jnp.where(sub == 0, -jnp.inf, x)
    m = jnp.max(xm, axis=0)
    pred21 = jnp.min(jnp.where(xm == m[None, :], sub * _C, _C * _C), axis=0)
    out_ref[...] = pred21 + gt_ref[...]


def _compute_bins(xt, gt):
    n = xt.shape[1]
    nblk = n // _BL
    return pl.pallas_call(
        _bins_body,
        grid=(nblk,),
        in_specs=[
            pl.BlockSpec((_C, _BL), lambda i: (0, i)),
            pl.BlockSpec((_BL,), lambda i: (i,)),
        ],
        out_specs=pl.BlockSpec((_BL,), lambda i: (i,)),
        out_shape=jax.ShapeDtypeStruct((n,), jnp.int32),
    )(xt, gt)


@functools.cache
def _make_hist(n):
    per_w = n // _NW
    n_chunks = per_w // _CHUNK
    mesh = plsc.VectorSubcoreMesh(core_axis_name="c", subcore_axis_name="s")

    @functools.partial(
        pl.kernel,
        out_type=jax.ShapeDtypeStruct((_NW, _NBINS), jnp.float32),
        mesh=mesh,
        compiler_params=pltpu.CompilerParams(needs_layout_passes=False),
        scratch_types=[
            pltpu.VMEM((2, _CHUNK), jnp.int32),
            pltpu.VMEM((_LANES * _NBINS,), jnp.float32),
            pltpu.VMEM((_NBINS,), jnp.float32),
            pltpu.SemaphoreType.DMA,
            pltpu.SemaphoreType.DMA,
        ],
    )
    def hist(bins_hbm, out_hbm, chunk_v, hist_v, part_v, sem0, sem1):
        cid = lax.axis_index("c")
        sid = lax.axis_index("s")
        wid = sid * 2 + cid
        base = wid * per_w
        lane = lax.iota(jnp.int32, _LANES)
        zeros16 = jnp.zeros((_LANES,), jnp.float32)
        ones16 = jnp.ones((_LANES,), jnp.float32)
        sems = (sem0, sem1)

        @plsc.parallel_loop(0, (_LANES * _NBINS) // _LANES, 1, unroll=8)
        def _zinit(i):
            hist_v[pl.ds(i * _LANES, _LANES)] = zeros16

        _U = 8
        handles = [None, None]
        handles[0] = pltpu.async_copy(
            bins_hbm.at[pl.ds(base, _CHUNK)], chunk_v.at[0], sems[0])
        for t in range(n_chunks):
            tb = t % 2
            if t + 1 < n_chunks:
                nb = (t + 1) % 2
                handles[nb] = pltpu.async_copy(
                    bins_hbm.at[pl.ds(base + (t + 1) * _CHUNK, _CHUNK)],
                    chunk_v.at[nb], sems[nb])
            handles[tb].wait()

            @plsc.parallel_loop(0, _CHUNK // _LANES, 1, unroll=_U)
            def _inner(i, tb=tb):
                b = chunk_v[tb, pl.ds(i * _LANES, _LANES)]
                plsc.addupdate_scatter(hist_v, [b * _LANES + lane], ones16)

        gidx = lane * _LANES
        for jb in range(_NBINS // _LANES):
            acc = zeros16
            for l in range(_LANES):
                g = plsc.load_gather(hist_v,
                                     [gidx + (jb * _LANES * _LANES + l)])
                acc = acc + g
            part_v[pl.ds(jb * _LANES, _LANES)] = acc
        pltpu.sync_copy(part_v, out_hbm.at[wid])

    return hist


def kernel(pred_logits, gt_labels):
    xt = jnp.transpose(pred_logits)
    bins = _compute_bins(xt, gt_labels.astype(jnp.int32))
    parts = _make_hist(pred_logits.shape[0])(bins)
    return parts.sum(axis=0)[: _C * _C].reshape(_C, _C)


# speedup vs baseline: 26.5182x; 1.0007x over previous
import functools

import jax
import jax.numpy as jnp
from jax import lax
from jax.experimental import pallas as pl
from jax.experimental.pallas import tpu as pltpu
from jax.experimental.pallas import tpu_sc as plsc

_C = 21
_NBINS = 448
_BL = 65536
_NW = 32
_CHUNK = 16384
_LANES = 16


def _bins_body(xt_ref, gt_ref, out_ref):
    x = xt_ref[...]
    sub = lax.broadcasted_iota(jnp.int32, x.shape, 0)
    xm = jnp.where(sub == 0, -jnp.inf, x)
    m = jnp.max(xm, axis=0)
    pred21 = jnp.min(jnp.where(xm == m[None, :], sub * _C, _C * _C), axis=0)
    out_ref[...] = pred21 + gt_ref[...]


def _compute_bins(xt, gt, half, nhalf):
    nblk = nhalf // _BL
    off = half * nblk
    return pl.pallas_call(
        _bins_body,
        grid=(nblk,),
        in_specs=[
            pl.BlockSpec((_C, _BL), lambda i: (0, i + off)),
            pl.BlockSpec((_BL,), lambda i: (i + off,)),
        ],
        out_specs=pl.BlockSpec((_BL,), lambda i: (i,)),
        out_shape=jax.ShapeDtypeStruct((nhalf,), jnp.int32),
    )(xt, gt)


@functools.cache
def _make_hist(n):
    per_w = n // _NW
    n_chunks = per_w // _CHUNK
    mesh = plsc.VectorSubcoreMesh(core_axis_name="c", subcore_axis_name="s")

    @functools.partial(
        pl.kernel,
        out_type=jax.ShapeDtypeStruct((_NW, _NBINS), jnp.float32),
        mesh=mesh,
        compiler_params=pltpu.CompilerParams(needs_layout_passes=False),
        scratch_types=[
            pltpu.VMEM((2, _CHUNK), jnp.int32),
            pltpu.VMEM((_LANES * _NBINS,), jnp.float32),
            pltpu.VMEM((_NBINS,), jnp.float32),
            pltpu.SemaphoreType.DMA,
            pltpu.SemaphoreType.DMA,
        ],
    )
    def hist(bins_hbm, out_hbm, chunk_v, hist_v, part_v, sem0, sem1):
        cid = lax.axis_index("c")
        sid = lax.axis_index("s")
        wid = sid * 2 + cid
        base = wid * per_w
        lane = lax.iota(jnp.int32, _LANES)
        zeros16 = jnp.zeros((_LANES,), jnp.float32)
        ones16 = jnp.ones((_LANES,), jnp.float32)
        sems = (sem0, sem1)

        @plsc.parallel_loop(0, (_LANES * _NBINS) // _LANES, 1, unroll=8)
        def _zinit(i):
            hist_v[pl.ds(i * _LANES, _LANES)] = zeros16

        _U = 8
        handles = [None, None]
        handles[0] = pltpu.async_copy(
            bins_hbm.at[pl.ds(base, _CHUNK)], chunk_v.at[0], sems[0])
        for t in range(n_chunks):
            tb = t % 2
            if t + 1 < n_chunks:
                nb = (t + 1) % 2
                handles[nb] = pltpu.async_copy(
                    bins_hbm.at[pl.ds(base + (t + 1) * _CHUNK, _CHUNK)],
                    chunk_v.at[nb], sems[nb])
            handles[tb].wait()

            @plsc.parallel_loop(0, _CHUNK // _LANES, 1, unroll=_U)
            def _inner(i, tb=tb):
                b = chunk_v[tb, pl.ds(i * _LANES, _LANES)]
                plsc.addupdate_scatter(hist_v, [b * _LANES + lane], ones16)

        gidx = lane * _LANES
        for jb in range(_NBINS // _LANES):
            acc = zeros16
            for l in range(_LANES):
                g = plsc.load_gather(hist_v,
                                     [gidx + (jb * _LANES * _LANES + l)])
                acc = acc + g
            part_v[pl.ds(jb * _LANES, _LANES)] = acc
        pltpu.sync_copy(part_v, out_hbm.at[wid])

    return hist


def kernel(pred_logits, gt_labels):
    n = pred_logits.shape[0]
    nhalf = n // 2
    xt = jnp.transpose(pred_logits)
    gt = gt_labels.astype(jnp.int32)
    hist = _make_hist(nhalf)
    bins0 = _compute_bins(xt, gt, 0, nhalf)
    parts0 = hist(bins0)
    bins1 = _compute_bins(xt, gt, 1, nhalf)
    parts1 = hist(bins1)
    parts = parts0 + parts1
    return parts.sum(axis=0)[: _C * _C].reshape(_C, _C)


# speedup vs baseline: 27.9413x; 1.0537x over previous
import functools

import jax
import jax.numpy as jnp
from jax import lax
from jax.experimental import pallas as pl
from jax.experimental.pallas import tpu as pltpu
from jax.experimental.pallas import tpu_sc as plsc

_C = 21
_NBINS = 448
_BL = 131072
_NW = 32
_CHUNK = 16384
_LANES = 16


def _bins_body(xt_ref, gt_ref, out_ref):
    x = xt_ref[...]
    sub = lax.broadcasted_iota(jnp.int32, x.shape, 0)
    xm = jnp.where(sub == 0, -jnp.inf, x)
    m = jnp.max(xm, axis=0)
    pred21 = jnp.min(jnp.where(xm == m[None, :], sub * _C, _C * _C), axis=0)
    out_ref[...] = pred21 + gt_ref[...]


def _compute_bins(xt, gt, half, nhalf):
    nblk = nhalf // _BL
    off = half * nblk
    return pl.pallas_call(
        _bins_body,
        grid=(nblk,),
        in_specs=[
            pl.BlockSpec((_C, _BL), lambda i: (0, i + off)),
            pl.BlockSpec((_BL,), lambda i: (i + off,)),
        ],
        out_specs=pl.BlockSpec((_BL,), lambda i: (i,)),
        out_shape=jax.ShapeDtypeStruct((nhalf,), jnp.int32),
    )(xt, gt)


@functools.cache
def _make_hist(n):
    per_w = n // _NW
    n_chunks = per_w // _CHUNK
    mesh = plsc.VectorSubcoreMesh(core_axis_name="c", subcore_axis_name="s")

    @functools.partial(
        pl.kernel,
        out_type=jax.ShapeDtypeStruct((_NW, _NBINS), jnp.float32),
        mesh=mesh,
        compiler_params=pltpu.CompilerParams(needs_layout_passes=False),
        scratch_types=[
            pltpu.VMEM((2, _CHUNK), jnp.int32),
            pltpu.VMEM((_LANES * _NBINS,), jnp.float32),
            pltpu.VMEM((_NBINS,), jnp.float32),
            pltpu.SemaphoreType.DMA,
            pltpu.SemaphoreType.DMA,
        ],
    )
    def hist(bins_hbm, out_hbm, chunk_v, hist_v, part_v, sem0, sem1):
        cid = lax.axis_index("c")
        sid = lax.axis_index("s")
        wid = sid * 2 + cid
        base = wid * per_w
        lane = lax.iota(jnp.int32, _LANES)
        zeros16 = jnp.zeros((_LANES,), jnp.float32)
        ones16 = jnp.ones((_LANES,), jnp.float32)
        sems = (sem0, sem1)

        @plsc.parallel_loop(0, (_LANES * _NBINS) // _LANES, 1, unroll=8)
        def _zinit(i):
            hist_v[pl.ds(i * _LANES, _LANES)] = zeros16

        _U = 8
        handles = [None, None]
        handles[0] = pltpu.async_copy(
            bins_hbm.at[pl.ds(base, _CHUNK)], chunk_v.at[0], sems[0])
        for t in range(n_chunks):
            tb = t % 2
            if t + 1 < n_chunks:
                nb = (t + 1) % 2
                handles[nb] = pltpu.async_copy(
                    bins_hbm.at[pl.ds(base + (t + 1) * _CHUNK, _CHUNK)],
                    chunk_v.at[nb], sems[nb])
            handles[tb].wait()

            @plsc.parallel_loop(0, _CHUNK // _LANES, 1, unroll=_U)
            def _inner(i, tb=tb):
                b = chunk_v[tb, pl.ds(i * _LANES, _LANES)]
                plsc.addupdate_scatter(hist_v, [b * _LANES + lane], ones16)

        gidx = lane * _LANES
        for jb in range(_NBINS // _LANES):
            acc = zeros16
            for l in range(_LANES):
                g = plsc.load_gather(hist_v,
                                     [gidx + (jb * _LANES * _LANES + l)])
                acc = acc + g
            part_v[pl.ds(jb * _LANES, _LANES)] = acc
        pltpu.sync_copy(part_v, out_hbm.at[wid])

    return hist


def kernel(pred_logits, gt_labels):
    n = pred_logits.shape[0]
    xt = jnp.transpose(pred_logits)
    gt = gt_labels.astype(jnp.int32)
    bins = _compute_bins(xt, gt, 0, n)
    parts = _make_hist(n)(bins)
    return parts.sum(axis=0)[: _C * _C].reshape(_C, _C)
